# baseline (speedup 1.0000x reference)
import functools

import jax
import jax.numpy as jnp
from jax import lax
from jax.experimental import pallas as pl
from jax.experimental.pallas import tpu as pltpu
from jax.experimental.pallas import tpu_sc as plsc

N = 10000
NP = 10240
E = 50000
EB = 96
NB = 36
EP = 16 * NB * EB
MB = 400
K1 = 1433
H1 = 1024
H2 = 512
W = 128
DO = 128


def _bf(x):
    return x.astype(jnp.bfloat16)


def _mm1_body(a_ref, w_ref, o_ref):
    y = lax.dot_general(_bf(a_ref[...]), w_ref[...],
                        dimension_numbers=(((0,), (0,)), ((), ())),
                        preferred_element_type=jnp.float32)
    for c in range(H1 // W):
        o_ref[c] = y[:, c * W:(c + 1) * W]


def _mm1(at, w):
    c = H1 // W
    mb = 512
    return pl.pallas_call(
        _mm1_body,
        grid=(pl.cdiv(N, mb),),
        in_specs=[
            pl.BlockSpec((K1, mb), lambda i: (0, i)),
            pl.BlockSpec((K1, H1), lambda i: (0, 0)),
        ],
        out_specs=pl.BlockSpec((c, mb, W), lambda i: (0, i, 0)),
        out_shape=jax.ShapeDtypeStruct((c, N, W), jnp.float32),
    )(at, w)


def _mm2_body(a_ref, b_ref, w_ref, o_ref):
    x = jnp.maximum(a_ref[...] + b_ref[...], 0.0)
    y = jnp.dot(_bf(x), w_ref[...], preferred_element_type=jnp.float32)
    for c in range(H2 // W):
        o_ref[c] = y[:, c * W:(c + 1) * W]


def _mm2(agg, b, w):
    c = H2 // W
    return pl.pallas_call(
        _mm2_body,
        grid=(N // MB,),
        in_specs=[
            pl.BlockSpec((MB, H1), lambda i: (i, 0)),
            pl.BlockSpec((1, H1), lambda i: (0, 0)),
            pl.BlockSpec((H1, H2), lambda i: (0, 0)),
        ],
        out_specs=pl.BlockSpec((c, MB, W), lambda i: (0, i, 0)),
        out_shape=jax.ShapeDtypeStruct((c, N, W), jnp.float32),
    )(agg, b, w)


def _mm3_body(a_ref, b2_ref, w_ref, b3_ref, o_ref):
    x = jnp.maximum(a_ref[...] + b2_ref[...], 0.0)
    y = jnp.dot(_bf(x), w_ref[...], preferred_element_type=jnp.float32)
    o_ref[...] = jnp.maximum(y + b3_ref[...], 0.0)


def _mm3(agg, b2, w, b3):
    return pl.pallas_call(
        _mm3_body,
        grid=(N // MB,),
        in_specs=[
            pl.BlockSpec((MB, H2), lambda i: (i, 0)),
            pl.BlockSpec((1, H2), lambda i: (0, 0)),
            pl.BlockSpec((H2, DO), lambda i: (0, 0)),
            pl.BlockSpec((1, DO), lambda i: (0, 0)),
        ],
        out_specs=pl.BlockSpec((MB, DO), lambda i: (i, 0)),
        out_shape=jax.ShapeDtypeStruct((N, DO), jnp.float32),
    )(agg, b2, w, b3)


def _make_agg(c_chunks):
    half = c_chunks // 2
    rows_t = NP // 16
    mesh = plsc.VectorSubcoreMesh(core_axis_name="c", subcore_axis_name="s")

    @functools.partial(
        pl.kernel,
        mesh=mesh,
        out_type=jax.ShapeDtypeStruct((NP, c_chunks * W), jnp.float32),
        scratch_types=[
            pltpu.VMEM_SHARED((NP, W), jnp.float32),
            pltpu.VMEM((NB, EB), jnp.int32),
            pltpu.VMEM((NB, EB), jnp.int32),
            pltpu.VMEM((EB, W), jnp.float32),
            pltpu.VMEM((EB, W), jnp.float32),
            pltpu.VMEM((EB, W), jnp.float32),
            pltpu.SemaphoreType.DMA,
            pltpu.SemaphoreType.DMA,
            pltpu.SemaphoreType.DMA,
            pltpu.SemaphoreType.DMA,
            pltpu.SemaphoreType.DMA,
            pltpu.SemaphoreType.DMA,
        ],
    )
    def agg(h, src, dst, zeros, out, slab, src_v, dst_v, rows0, rows1,
            rows2, g0, g1, g2, s0, s1, s2):
        bufs = (rows0, rows1, rows2)
        gsem = (g0, g1, g2)
        ssem = (s0, s1, s2)
        cid = lax.axis_index("c")
        sid = lax.axis_index("s")
        r0 = sid * rows_t
        pltpu.sync_copy(src.at[sid], src_v)
        pltpu.sync_copy(dst.at[sid], dst_v)

        def chunk_body(ci, carry):
            chunk = cid * half + ci
            pltpu.sync_copy(zeros.at[pl.ds(r0, rows_t)],
                            slab.at[pl.ds(r0, rows_t)])
            plsc.subcore_barrier()

            pltpu.async_copy(h.at[chunk].at[src_v.at[0]], bufs[0], gsem[0])

            def batch_body(o, carry2):
                for j in range(3):
                    b = o * 3 + j
                    jn = (j + 1) % 3

                    @pl.when(b + 1 < NB)
                    def _():
                        @pl.when(b >= 2)
                        def _():
                            pltpu.make_async_copy(
                                bufs[jn], slab.at[dst_v.at[b - 2]],
                                ssem[jn]).wait()

                        pltpu.async_copy(h.at[chunk].at[src_v.at[b + 1]],
                                         bufs[jn], gsem[jn])

                    pltpu.make_async_copy(h.at[chunk].at[src_v.at[b]],
                                          bufs[j], gsem[j]).wait()
                    pltpu.async_copy(bufs[j], slab.at[dst_v.at[b]],
                                     ssem[j], add=True)
                return carry2

            lax.fori_loop(0, NB // 3, batch_body, 0)
            for t in range(3):
                b = NB - 3 + t
                pltpu.make_async_copy(bufs[b % 3], slab.at[dst_v.at[b]],
                                      ssem[b % 3]).wait()
            plsc.subcore_barrier()
            pltpu.sync_copy(slab.at[pl.ds(r0, rows_t)],
                            out.at[pl.ds(r0, rows_t), pl.ds(chunk * W, W)])
            plsc.subcore_barrier()
            return carry

        lax.fori_loop(0, half, chunk_body, 0)

    return agg


_agg8 = _make_agg(8)
_agg4 = _make_agg(4)


def kernel(features, edge_index, W1, b1, W2, b2, W3, b3):
    f32 = jnp.float32
    bf16 = jnp.bfloat16
    W1p = jnp.zeros((K1, H1), bf16).at[:, :1000].set(_bf(W1))
    b1p = jnp.pad(b1, (0, H1 - 1000)).reshape(1, H1)
    W2p = jnp.zeros((H1, H2), bf16).at[:1000, :500].set(_bf(W2))
    b2p = jnp.pad(b2, (0, H2 - 500)).reshape(1, H2)
    W3p = jnp.zeros((H2, DO), bf16).at[:500, :].set(_bf(W3))
    b3p = b3.reshape(1, DO)

    npad = EP - E
    pad_i = jnp.arange(npad, dtype=jnp.int32)
    srcp = jnp.concatenate([edge_index[0], pad_i % N]).reshape(16, NB, EB)
    dstp = jnp.concatenate([edge_index[1], N + pad_i % (NP - N)]
                           ).reshape(16, NB, EB)
    zeros = jnp.zeros((NP, W), f32)

    h1 = _mm1(features.T, W1p)
    agg1 = _agg8(h1, srcp, dstp, zeros)
    h2 = _mm2(agg1, b1p, W2p)
    agg2 = _agg4(h2, srcp, dstp, zeros)
    return _mm3(agg2, b2p, W3p, b3p)


# speedup vs baseline: 1.0274x; 1.0274x over previous
import functools

import jax
import jax.numpy as jnp
from jax import lax
from jax.experimental import pallas as pl
from jax.experimental.pallas import tpu as pltpu
from jax.experimental.pallas import tpu_sc as plsc

N = 10000
NP = 10240
E = 50000
EB = 128
NB = 26
EP = 16 * NB * EB
MB = 400
K1 = 1433
H1 = 1024
H2 = 512
W = 128
DO = 128


def _bf(x):
    return x.astype(jnp.bfloat16)


def _mm1_body(a_ref, w_ref, o_ref):
    y = lax.dot_general(_bf(a_ref[...]), w_ref[...],
                        dimension_numbers=(((0,), (0,)), ((), ())),
                        preferred_element_type=jnp.float32)
    for c in range(H1 // W):
        o_ref[c] = y[:, c * W:(c + 1) * W]


def _mm1(at, w):
    c = H1 // W
    mb = 512
    return pl.pallas_call(
        _mm1_body,
        grid=(pl.cdiv(N, mb),),
        in_specs=[
            pl.BlockSpec((K1, mb), lambda i: (0, i)),
            pl.BlockSpec((K1, H1), lambda i: (0, 0)),
        ],
        out_specs=pl.BlockSpec((c, mb, W), lambda i: (0, i, 0)),
        out_shape=jax.ShapeDtypeStruct((c, N, W), jnp.float32),
    )(at, w)


def _mm2_body(a_ref, b_ref, w_ref, o_ref):
    x = jnp.maximum(a_ref[...] + b_ref[...], 0.0)
    y = jnp.dot(_bf(x), w_ref[...], preferred_element_type=jnp.float32)
    for c in range(H2 // W):
        o_ref[c] = y[:, c * W:(c + 1) * W]


def _mm2(agg, b, w):
    c = H2 // W
    return pl.pallas_call(
        _mm2_body,
        grid=(N // MB,),
        in_specs=[
            pl.BlockSpec((MB, H1), lambda i: (i, 0)),
            pl.BlockSpec((1, H1), lambda i: (0, 0)),
            pl.BlockSpec((H1, H2), lambda i: (0, 0)),
        ],
        out_specs=pl.BlockSpec((c, MB, W), lambda i: (0, i, 0)),
        out_shape=jax.ShapeDtypeStruct((c, N, W), jnp.float32),
    )(agg, b, w)


def _mm3_body(a_ref, b2_ref, w_ref, b3_ref, o_ref):
    x = jnp.maximum(a_ref[...] + b2_ref[...], 0.0)
    y = jnp.dot(_bf(x), w_ref[...], preferred_element_type=jnp.float32)
    o_ref[...] = jnp.maximum(y + b3_ref[...], 0.0)


def _mm3(agg, b2, w, b3):
    return pl.pallas_call(
        _mm3_body,
        grid=(N // MB,),
        in_specs=[
            pl.BlockSpec((MB, H2), lambda i: (i, 0)),
            pl.BlockSpec((1, H2), lambda i: (0, 0)),
            pl.BlockSpec((H2, DO), lambda i: (0, 0)),
            pl.BlockSpec((1, DO), lambda i: (0, 0)),
        ],
        out_specs=pl.BlockSpec((MB, DO), lambda i: (i, 0)),
        out_shape=jax.ShapeDtypeStruct((N, DO), jnp.float32),
    )(agg, b2, w, b3)


def _make_agg(c_chunks):
    half = c_chunks // 2
    rows_t = NP // 16
    mesh = plsc.VectorSubcoreMesh(core_axis_name="c", subcore_axis_name="s")

    @functools.partial(
        pl.kernel,
        mesh=mesh,
        out_type=jax.ShapeDtypeStruct((NP, c_chunks * W), jnp.float32),
        scratch_types=[
            pltpu.VMEM_SHARED((NP, W), jnp.float32),
            pltpu.VMEM((NB, EB), jnp.int32),
            pltpu.VMEM((NB, EB), jnp.int32),
            pltpu.VMEM((EB, W), jnp.float32),
            pltpu.VMEM((EB, W), jnp.float32),
            pltpu.SemaphoreType.DMA,
            pltpu.SemaphoreType.DMA,
        ],
    )
    def agg(h, src, dst, zeros, out, slab, src_v, dst_v, rows0, rows1,
            sem0, sem1):
        bufs = (rows0, rows1)
        sems = (sem0, sem1)
        cid = lax.axis_index("c")
        sid = lax.axis_index("s")
        r0 = sid * rows_t
        pltpu.sync_copy(src.at[sid], src_v)
        pltpu.sync_copy(dst.at[sid], dst_v)
        pltpu.async_copy(h.at[cid * half].at[src_v.at[0]], bufs[0], sems[0])

        def chunk_body(ci, carry):
            chunk = cid * half + ci
            pltpu.sync_copy(zeros.at[pl.ds(r0, rows_t)],
                            slab.at[pl.ds(r0, rows_t)])
            plsc.subcore_barrier()

            def batch_body(o, carry2):
                for j in range(2):
                    b = o + j

                    @pl.when(b + 1 < NB)
                    def _():
                        pltpu.async_copy(h.at[chunk].at[src_v.at[b + 1]],
                                         bufs[1 - j], sems[1 - j])

                    pltpu.make_async_copy(h.at[chunk].at[src_v.at[b]],
                                          bufs[j], sems[j]).wait()
                    pltpu.sync_copy(bufs[j], slab.at[dst_v.at[b]], add=True)
                return carry2

            lax.fori_loop(0, NB // 2, lambda o, c: batch_body(o * 2, c), 0)

            @pl.when(ci + 1 < half)
            def _():
                pltpu.async_copy(h.at[chunk + 1].at[src_v.at[0]],
                                 bufs[0], sems[0])

            plsc.subcore_barrier()
            pltpu.sync_copy(slab.at[pl.ds(r0, rows_t)],
                            out.at[pl.ds(r0, rows_t), pl.ds(chunk * W, W)])
            plsc.subcore_barrier()
            return carry

        lax.fori_loop(0, half, chunk_body, 0)

    return agg


_agg8 = _make_agg(8)
_agg4 = _make_agg(4)


def kernel(features, edge_index, W1, b1, W2, b2, W3, b3):
    f32 = jnp.float32
    bf16 = jnp.bfloat16
    W1p = jnp.zeros((K1, H1), bf16).at[:, :1000].set(_bf(W1))
    b1p = jnp.pad(b1, (0, H1 - 1000)).reshape(1, H1)
    W2p = jnp.zeros((H1, H2), bf16).at[:1000, :500].set(_bf(W2))
    b2p = jnp.pad(b2, (0, H2 - 500)).reshape(1, H2)
    W3p = jnp.zeros((H2, DO), bf16).at[:500, :].set(_bf(W3))
    b3p = b3.reshape(1, DO)

    npad = EP - E
    pad_i = jnp.arange(npad, dtype=jnp.int32)
    srcp = jnp.concatenate([edge_index[0], pad_i % N]).reshape(16, NB, EB)
    dstp = jnp.concatenate([edge_index[1], N + pad_i % (NP - N)]
                           ).reshape(16, NB, EB)
    zeros = jnp.zeros((NP, W), f32)

    h1 = _mm1(features.T, W1p)
    agg1 = _agg8(h1, srcp, dstp, zeros)
    h2 = _mm2(agg1, b1p, W2p)
    agg2 = _agg4(h2, srcp, dstp, zeros)
    return _mm3(agg2, b2p, W3p, b3p)


# speedup vs baseline: 1.0394x; 1.0116x over previous
import functools

import jax
import jax.numpy as jnp
from jax import lax
from jax.experimental import pallas as pl
from jax.experimental.pallas import tpu as pltpu
from jax.experimental.pallas import tpu_sc as plsc

N = 10000
NP = 10240
E = 50000
EB = 128
NB = 26
EP = 16 * NB * EB
MB = 400
K1 = 1433
H1 = 1024
H2 = 512
W = 128
DO = 128


def _bf(x):
    return x.astype(jnp.bfloat16)


def _mm1_body(a_ref, w_ref, o_ref):
    y = lax.dot_general(_bf(a_ref[...]), w_ref[...],
                        dimension_numbers=(((0,), (0,)), ((), ())),
                        preferred_element_type=jnp.float32)
    for c in range(H1 // W):
        o_ref[c] = y[:, c * W:(c + 1) * W]


def _mm1(at, w):
    c = H1 // W
    mb = 1024
    return pl.pallas_call(
        _mm1_body,
        grid=(pl.cdiv(N, mb),),
        in_specs=[
            pl.BlockSpec((K1, mb), lambda i: (0, i)),
            pl.BlockSpec((K1, H1), lambda i: (0, 0)),
        ],
        out_specs=pl.BlockSpec((c, mb, W), lambda i: (0, i, 0)),
        out_shape=jax.ShapeDtypeStruct((c, N, W), jnp.float32),
    )(at, w)


def _mm2_body(a_ref, b_ref, w_ref, o_ref):
    x = jnp.maximum(a_ref[...] + b_ref[...], 0.0)
    y = jnp.dot(_bf(x), w_ref[...], preferred_element_type=jnp.float32)
    for c in range(H2 // W):
        o_ref[c] = y[:, c * W:(c + 1) * W]


def _mm2(agg, b, w):
    c = H2 // W
    return pl.pallas_call(
        _mm2_body,
        grid=(N // MB,),
        in_specs=[
            pl.BlockSpec((MB, H1), lambda i: (i, 0)),
            pl.BlockSpec((1, H1), lambda i: (0, 0)),
            pl.BlockSpec((H1, H2), lambda i: (0, 0)),
        ],
        out_specs=pl.BlockSpec((c, MB, W), lambda i: (0, i, 0)),
        out_shape=jax.ShapeDtypeStruct((c, N, W), jnp.float32),
    )(agg, b, w)


def _mm3_body(a_ref, b2_ref, w_ref, b3_ref, o_ref):
    x = jnp.maximum(a_ref[...] + b2_ref[...], 0.0)
    y = jnp.dot(_bf(x), w_ref[...], preferred_element_type=jnp.float32)
    o_ref[...] = jnp.maximum(y + b3_ref[...], 0.0)


def _mm3(agg, b2, w, b3):
    return pl.pallas_call(
        _mm3_body,
        grid=(N // MB,),
        in_specs=[
            pl.BlockSpec((MB, H2), lambda i: (i, 0)),
            pl.BlockSpec((1, H2), lambda i: (0, 0)),
            pl.BlockSpec((H2, DO), lambda i: (0, 0)),
            pl.BlockSpec((1, DO), lambda i: (0, 0)),
        ],
        out_specs=pl.BlockSpec((MB, DO), lambda i: (i, 0)),
        out_shape=jax.ShapeDtypeStruct((N, DO), jnp.float32),
    )(agg, b2, w, b3)


def _make_agg(c_chunks):
    half = c_chunks // 2
    rows_t = NP // 16
    mesh = plsc.VectorSubcoreMesh(core_axis_name="c", subcore_axis_name="s")

    @functools.partial(
        pl.kernel,
        mesh=mesh,
        out_type=jax.ShapeDtypeStruct((NP, c_chunks * W), jnp.float32),
        scratch_types=[
            pltpu.VMEM_SHARED((NP, W), jnp.float32),
            pltpu.VMEM((NB, EB), jnp.int32),
            pltpu.VMEM((NB, EB), jnp.int32),
            pltpu.VMEM((EB, W), jnp.float32),
            pltpu.VMEM((EB, W), jnp.float32),
            pltpu.SemaphoreType.DMA,
            pltpu.SemaphoreType.DMA,
        ],
    )
    def agg(h, src, dst, zeros, out, slab, src_v, dst_v, rows0, rows1,
            sem0, sem1):
        bufs = (rows0, rows1)
        sems = (sem0, sem1)
        cid = lax.axis_index("c")
        sid = lax.axis_index("s")
        r0 = sid * rows_t
        pltpu.sync_copy(src.at[sid], src_v)
        pltpu.sync_copy(dst.at[sid], dst_v)
        pltpu.async_copy(h.at[cid * half].at[src_v.at[0]], bufs[0], sems[0])

        def chunk_body(ci, carry):
            chunk = cid * half + ci
            pltpu.sync_copy(zeros.at[pl.ds(r0, rows_t)],
                            slab.at[pl.ds(r0, rows_t)])
            plsc.subcore_barrier()

            def batch_body(o, carry2):
                for j in range(2):
                    b = o + j

                    @pl.when(b + 1 < NB)
                    def _():
                        pltpu.async_copy(h.at[chunk].at[src_v.at[b + 1]],
                                         bufs[1 - j], sems[1 - j])

                    pltpu.make_async_copy(h.at[chunk].at[src_v.at[b]],
                                          bufs[j], sems[j]).wait()
                    pltpu.sync_copy(bufs[j], slab.at[dst_v.at[b]], add=True)
                return carry2

            lax.fori_loop(0, NB // 2, lambda o, c: batch_body(o * 2, c), 0)

            @pl.when(ci + 1 < half)
            def _():
                pltpu.async_copy(h.at[chunk + 1].at[src_v.at[0]],
                                 bufs[0], sems[0])

            plsc.subcore_barrier()
            pltpu.sync_copy(slab.at[pl.ds(r0, rows_t)],
                            out.at[pl.ds(r0, rows_t), pl.ds(chunk * W, W)])
            plsc.subcore_barrier()
            return carry

        lax.fori_loop(0, half, chunk_body, 0)

    return agg


_agg8 = _make_agg(8)
_agg4 = _make_agg(4)


def kernel(features, edge_index, W1, b1, W2, b2, W3, b3):
    f32 = jnp.float32
    bf16 = jnp.bfloat16
    W1p = jnp.zeros((K1, H1), bf16).at[:, :1000].set(_bf(W1))
    b1p = jnp.pad(b1, (0, H1 - 1000)).reshape(1, H1)
    W2p = jnp.zeros((H1, H2), bf16).at[:1000, :500].set(_bf(W2))
    b2p = jnp.pad(b2, (0, H2 - 500)).reshape(1, H2)
    W3p = jnp.zeros((H2, DO), bf16).at[:500, :].set(_bf(W3))
    b3p = b3.reshape(1, DO)

    npad = EP - E
    pad_i = jnp.arange(npad, dtype=jnp.int32)
    srcp = jnp.concatenate([edge_index[0], pad_i % N]).reshape(16, NB, EB)
    dstp = jnp.concatenate([edge_index[1], N + pad_i % (NP - N)]
                           ).reshape(16, NB, EB)
    zeros = jnp.zeros((NP, W), f32)

    h1 = _mm1(features.T, W1p)
    agg1 = _agg8(h1, srcp, dstp, zeros)
    h2 = _mm2(agg1, b1p, W2p)
    agg2 = _agg4(h2, srcp, dstp, zeros)
    return _mm3(agg2, b2p, W3p, b3p)


# speedup vs baseline: 1.0443x; 1.0048x over previous
import functools

import jax
import jax.numpy as jnp
from jax import lax
from jax.experimental import pallas as pl
from jax.experimental.pallas import tpu as pltpu
from jax.experimental.pallas import tpu_sc as plsc

N = 10000
NP = 10240
E = 50000
EB = 128
NB = 26
EP = 16 * NB * EB
MB = 400
K1 = 1433
H1 = 1024
H2 = 512
W = 128
DO = 128


def _bf(x):
    return x.astype(jnp.bfloat16)


def _mm1_body(a_ref, w_ref, o_ref):
    y = lax.dot_general(_bf(a_ref[...]), w_ref[...],
                        dimension_numbers=(((0,), (0,)), ((), ())),
                        preferred_element_type=jnp.float32)
    for c in range(H1 // W):
        o_ref[c] = y[:, c * W:(c + 1) * W]


def _mm1(at, w):
    c = H1 // W
    mb = 2048
    return pl.pallas_call(
        _mm1_body,
        grid=(pl.cdiv(N, mb),),
        in_specs=[
            pl.BlockSpec((K1, mb), lambda i: (0, i)),
            pl.BlockSpec((K1, H1), lambda i: (0, 0)),
        ],
        out_specs=pl.BlockSpec((c, mb, W), lambda i: (0, i, 0)),
        out_shape=jax.ShapeDtypeStruct((c, N, W), jnp.float32),
    )(at, w)


def _mm2_body(a_ref, b_ref, w_ref, o_ref):
    x = jnp.maximum(a_ref[...] + b_ref[...], 0.0)
    y = jnp.dot(_bf(x), w_ref[...], preferred_element_type=jnp.float32)
    for c in range(H2 // W):
        o_ref[c] = y[:, c * W:(c + 1) * W]


def _mm2(agg, b, w):
    c = H2 // W
    return pl.pallas_call(
        _mm2_body,
        grid=(N // MB,),
        in_specs=[
            pl.BlockSpec((MB, H1), lambda i: (i, 0)),
            pl.BlockSpec((1, H1), lambda i: (0, 0)),
            pl.BlockSpec((H1, H2), lambda i: (0, 0)),
        ],
        out_specs=pl.BlockSpec((c, MB, W), lambda i: (0, i, 0)),
        out_shape=jax.ShapeDtypeStruct((c, N, W), jnp.float32),
    )(agg, b, w)


def _mm3_body(a_ref, b2_ref, w_ref, b3_ref, o_ref):
    x = jnp.maximum(a_ref[...] + b2_ref[...], 0.0)
    y = jnp.dot(_bf(x), w_ref[...], preferred_element_type=jnp.float32)
    o_ref[...] = jnp.maximum(y + b3_ref[...], 0.0)


def _mm3(agg, b2, w, b3):
    return pl.pallas_call(
        _mm3_body,
        grid=(N // MB,),
        in_specs=[
            pl.BlockSpec((MB, H2), lambda i: (i, 0)),
            pl.BlockSpec((1, H2), lambda i: (0, 0)),
            pl.BlockSpec((H2, DO), lambda i: (0, 0)),
            pl.BlockSpec((1, DO), lambda i: (0, 0)),
        ],
        out_specs=pl.BlockSpec((MB, DO), lambda i: (i, 0)),
        out_shape=jax.ShapeDtypeStruct((N, DO), jnp.float32),
    )(agg, b2, w, b3)


def _make_agg(c_chunks):
    half = c_chunks // 2
    rows_t = NP // 16
    mesh = plsc.VectorSubcoreMesh(core_axis_name="c", subcore_axis_name="s")

    @functools.partial(
        pl.kernel,
        mesh=mesh,
        out_type=jax.ShapeDtypeStruct((NP, c_chunks * W), jnp.float32),
        scratch_types=[
            pltpu.VMEM_SHARED((NP, W), jnp.float32),
            pltpu.VMEM((NB, EB), jnp.int32),
            pltpu.VMEM((NB, EB), jnp.int32),
            pltpu.VMEM((EB, W), jnp.float32),
            pltpu.VMEM((EB, W), jnp.float32),
            pltpu.SemaphoreType.DMA,
            pltpu.SemaphoreType.DMA,
        ],
    )
    def agg(h, src, dst, zeros, out, slab, src_v, dst_v, rows0, rows1,
            sem0, sem1):
        bufs = (rows0, rows1)
        sems = (sem0, sem1)
        cid = lax.axis_index("c")
        sid = lax.axis_index("s")
        r0 = sid * rows_t
        pltpu.sync_copy(src.at[sid], src_v)
        pltpu.sync_copy(dst.at[sid], dst_v)
        pltpu.async_copy(h.at[cid * half].at[src_v.at[0]], bufs[0], sems[0])

        def chunk_body(ci, carry):
            chunk = cid * half + ci
            pltpu.sync_copy(zeros.at[pl.ds(r0, rows_t)],
                            slab.at[pl.ds(r0, rows_t)])
            plsc.subcore_barrier()

            def batch_body(o, carry2):
                for j in range(2):
                    b = o + j

                    @pl.when(b + 1 < NB)
                    def _():
                        pltpu.async_copy(h.at[chunk].at[src_v.at[b + 1]],
                                         bufs[1 - j], sems[1 - j])

                    pltpu.make_async_copy(h.at[chunk].at[src_v.at[b]],
                                          bufs[j], sems[j]).wait()
                    pltpu.sync_copy(bufs[j], slab.at[dst_v.at[b]], add=True)
                return carry2

            lax.fori_loop(0, NB // 2, lambda o, c: batch_body(o * 2, c), 0)

            @pl.when(ci + 1 < half)
            def _():
                pltpu.async_copy(h.at[chunk + 1].at[src_v.at[0]],
                                 bufs[0], sems[0])

            plsc.subcore_barrier()
            pltpu.sync_copy(slab.at[pl.ds(r0, rows_t)],
                            out.at[pl.ds(r0, rows_t), pl.ds(chunk * W, W)])
            plsc.subcore_barrier()
            return carry

        lax.fori_loop(0, half, chunk_body, 0)

    return agg


_agg8 = _make_agg(8)
_agg4 = _make_agg(4)


def kernel(features, edge_index, W1, b1, W2, b2, W3, b3):
    f32 = jnp.float32
    bf16 = jnp.bfloat16
    W1p = jnp.zeros((K1, H1), bf16).at[:, :1000].set(_bf(W1))
    b1p = jnp.pad(b1, (0, H1 - 1000)).reshape(1, H1)
    W2p = jnp.zeros((H1, H2), bf16).at[:1000, :500].set(_bf(W2))
    b2p = jnp.pad(b2, (0, H2 - 500)).reshape(1, H2)
    W3p = jnp.zeros((H2, DO), bf16).at[:500, :].set(_bf(W3))
    b3p = b3.reshape(1, DO)

    npad = EP - E
    pad_i = jnp.arange(npad, dtype=jnp.int32)
    srcp = jnp.concatenate([edge_index[0], pad_i % N]).reshape(16, NB, EB)
    dstp = jnp.concatenate([edge_index[1], N + pad_i % (NP - N)]
                           ).reshape(16, NB, EB)
    zeros = jnp.zeros((NP, W), f32)

    h1 = _mm1(features.T, W1p)
    agg1 = _agg8(h1, srcp, dstp, zeros)
    h2 = _mm2(agg1, b1p, W2p)
    agg2 = _agg4(h2, srcp, dstp, zeros)
    return _mm3(agg2, b2p, W3p, b3p)
